# baseline (speedup 1.0000x reference)
import jax
import jax.numpy as jnp
from jax import lax
from jax.experimental import pallas as pl
from jax.experimental.pallas import tpu as pltpu
from jax.experimental.pallas import tpu_sc as plsc

H = 512
W = 512
B = 16
P = H * W
R = 10.0
MAGIC = 1.5 * (2.0 ** 23)

NC = 2
NS = 16
L = 16

BATCHES_PER_CORE = B // NC
PASS_BATCHES = 2
NPASS = BATCHES_PER_CORE // PASS_BATCHES
TILES_PER_BATCH = NS // PASS_BATCHES
ROWS_PER_TILE = H // TILES_PER_BATCH
RCH = 8
CH = RCH * W
NCHUNK = ROWS_PER_TILE // RCH
HIST = PASS_BATCHES * P
SLICE = HIST // NS
LH = ROWS_PER_TILE * W
OC = 8192
FLUSH_AT = OC - 8 * L
ZB = 8192
DSEC = 8192


def _body(w_hbm, off_hbm, out_hbm, hist,
          dx_a, dx_b, dy_a, dy_b, w_a, w_b,
          lhist, oob_idx, oob_w, dstage, zero_v, xf_buf,
          sin_a, sin_b):
    c = lax.axis_index("c")
    s = lax.axis_index("s")
    b_in_pass = s // TILES_PER_BATCH
    part = s % TILES_PER_BATCH
    band0 = part * ROWS_PER_TILE
    band_flat = band0 * W
    myslice = s * SLICE
    lanes = lax.iota(jnp.int32, L)
    zero16i = jnp.zeros((L,), jnp.int32)
    zero16f = jnp.zeros((L,), jnp.float32)

    dx_r = (dx_a, dx_b)
    dy_r = (dy_a, dy_b)
    w_r = (w_a, w_b)
    sin_r = (sin_a, sin_b)

    def zinit(i, carry):
        zero_v[pl.ds(i * L, L)] = zero16f
        return carry

    lax.fori_loop(0, ZB // L, zinit, 0)

    def xinit(j, carry):
        xf_buf[pl.ds(j * L, L)] = (lanes + j * L).astype(jnp.float32)
        return carry

    lax.fori_loop(0, W // L, xinit, 0)

    def lhinit(i, carry):
        lhist[pl.ds(i * L, L)] = zero16f
        return carry

    lax.fori_loop(0, LH // L, lhinit, 0)

    def oob_zero(i, carry):
        oob_idx[pl.ds(i * L, L)] = zero16i
        oob_w[pl.ds(i * L, L)] = zero16f
        return carry

    lax.fori_loop(0, OC // L, oob_zero, 0)

    def pass_body(pidx, pcarry):
        b_global = c * BATCHES_PER_CORE + pidx * PASS_BATCHES + b_in_pass
        for q in range(SLICE // ZB):
            pltpu.sync_copy(zero_v, hist.at[pl.ds(myslice + q * ZB, ZB)])
        plsc.subcore_barrier()

        hist_off = b_in_pass * P

        def fire_inputs(ci, sl):
            y0 = band0 + ci * RCH
            pltpu.async_copy(
                w_hbm.at[b_global, pl.ds(y0, RCH), :], w_r[sl], sin_r[sl])
            pltpu.async_copy(
                off_hbm.at[2 * b_global, pl.ds(y0, RCH), :], dx_r[sl],
                sin_r[sl])
            pltpu.async_copy(
                off_hbm.at[2 * b_global + 1, pl.ds(y0, RCH), :], dy_r[sl],
                sin_r[sl])

        def wait_inputs(sl):
            src = w_hbm.at[b_global, pl.ds(0, RCH), :]
            pltpu.make_async_copy(src, w_r[sl], sin_r[sl]).wait()
            pltpu.make_async_copy(src, dx_r[sl], sin_r[sl]).wait()
            pltpu.make_async_copy(src, dy_r[sl], sin_r[sl]).wait()

        def compute_chunk(ci, sl, offset):
            y0 = band0 + ci * RCH
            dx_cur = dx_r[sl]
            dy_cur = dy_r[sl]
            w_cur = w_r[sl]
            yfs = [(y0 + r).astype(jnp.float32) for r in range(RCH)]

            def vec_body(j, off):
                pred = jnp.max(off) > FLUSH_AT

                @pl.when(pred)
                def _flush():
                    pltpu.sync_copy(oob_w, hist.at[oob_idx], add=True)
                    lax.fori_loop(0, OC // L, oob_zero, 0)

                off = jnp.where(pred, 0, off)
                xf = xf_buf[pl.ds(j * L, L)]
                for r in range(RCH):
                    dxv = dx_cur[r, pl.ds(j * L, L)]
                    dyv = dy_cur[r, pl.ds(j * L, L)]
                    wv = w_cur[r, pl.ds(j * L, L)]
                    vx = (xf + R * dxv + MAGIC) - MAGIC
                    vy = (yfs[r] + R * dyv + MAGIC) - MAGIC
                    vx = jnp.minimum(jnp.maximum(vx, 0.0), W - 1.0)
                    vy = jnp.minimum(jnp.maximum(vy, 0.0), H - 1.0)
                    vyi = vy.astype(jnp.int32)
                    vxi = vx.astype(jnp.int32)
                    idxfull = lax.shift_left(vyi, 9) + vxi
                    inb = jnp.logical_and(vyi >= band0,
                                          vyi < band0 + ROWS_PER_TILE)
                    land = lax.bitwise_and(idxfull - band_flat, LH - 1)
                    plsc.addupdate_scatter(lhist, [land], wv, mask=inb)
                    oob = jnp.logical_not(inb)
                    cum = plsc.cumsum(oob.astype(jnp.int32))
                    cnt = plsc.all_reduce_population_count(oob)
                    pos = lax.bitwise_and(off + cum - 1, OC - 1)
                    plsc.store_scatter(oob_idx, [pos], idxfull + hist_off,
                                       mask=oob)
                    plsc.store_scatter(oob_w, [pos], wv, mask=oob)
                    off = off + cnt
                return off

            return lax.fori_loop(0, W // L, vec_body, offset)

        fire_inputs(0, 0)

        def group_body(g, offset):
            e = 2 * g
            o = 2 * g + 1
            fire_inputs(o, 1)
            wait_inputs(0)
            offset = compute_chunk(e, 0, offset)

            @pl.when(o + 1 < NCHUNK)
            def _prefetch_even():
                fire_inputs(o + 1, 0)

            wait_inputs(1)
            offset = compute_chunk(o, 1, offset)
            return offset

        lax.fori_loop(0, NCHUNK // 2, group_body, zero16i)

        pltpu.sync_copy(oob_w, hist.at[oob_idx], add=True)
        lax.fori_loop(0, OC // L, oob_zero, 0)
        plsc.subcore_barrier()

        pass_out = (c * BATCHES_PER_CORE + pidx * PASS_BATCHES) * P
        for sec in range(SLICE // DSEC):
            pltpu.sync_copy(hist.at[pl.ds(myslice + sec * DSEC, DSEC)],
                            dstage)

            def addz(v, carry, sec=sec):
                o = sec * DSEC + v * L
                dstage[pl.ds(v * L, L)] = (dstage[pl.ds(v * L, L)]
                                           + lhist[pl.ds(o, L)])
                lhist[pl.ds(o, L)] = zero16f
                return carry

            lax.fori_loop(0, DSEC // L, addz, 0)
            pltpu.sync_copy(
                dstage,
                out_hbm.at[pl.ds(pass_out + myslice + sec * DSEC, DSEC)])
        return pcarry

    lax.fori_loop(0, NPASS, pass_body, 0)


def kernel(stem_keypoint_output, stem_offset_output):
    w3 = stem_keypoint_output.reshape(B, H, W)
    off3 = stem_offset_output.reshape(2 * B, H, W)
    mesh = plsc.VectorSubcoreMesh(core_axis_name="c", subcore_axis_name="s")
    out = pl.kernel(
        _body,
        out_type=jax.ShapeDtypeStruct((B * P,), jnp.float32),
        mesh=mesh,
        compiler_params=pltpu.CompilerParams(use_tc_tiling_on_sc=True,
                                             needs_layout_passes=False),
        scratch_types=[
            pltpu.VMEM_SHARED((HIST,), jnp.float32),
            pltpu.VMEM((RCH, W), jnp.float32),
            pltpu.VMEM((RCH, W), jnp.float32),
            pltpu.VMEM((RCH, W), jnp.float32),
            pltpu.VMEM((RCH, W), jnp.float32),
            pltpu.VMEM((RCH, W), jnp.float32),
            pltpu.VMEM((RCH, W), jnp.float32),
            pltpu.VMEM((LH,), jnp.float32),
            pltpu.VMEM((OC,), jnp.int32),
            pltpu.VMEM((OC,), jnp.float32),
            pltpu.VMEM((DSEC,), jnp.float32),
            pltpu.VMEM((ZB,), jnp.float32),
            pltpu.VMEM((W,), jnp.float32),
            pltpu.SemaphoreType.DMA,
            pltpu.SemaphoreType.DMA,
        ],
    )(w3, off3)
    return out.reshape(B, H, W)


# speedup vs baseline: 3.6961x; 3.6961x over previous
import jax
import jax.numpy as jnp
from jax import lax
from jax.experimental import pallas as pl
from jax.experimental.pallas import tpu as pltpu
from jax.experimental.pallas import tpu_sc as plsc

H = 512
W = 512
B = 16
P = H * W
R = 10.0
MAGIC = 1.5 * (2.0 ** 23)

NC = 2
NS = 16
L = 16

BATCHES_PER_CORE = B // NC
PASS_BATCHES = 4
NPASS = BATCHES_PER_CORE // PASS_BATCHES
TILES_PER_BATCH = NS // PASS_BATCHES
ROWS_PER_TILE = H // TILES_PER_BATCH
RCH = 8
CH = RCH * W
NCHUNK = ROWS_PER_TILE // RCH
SLOTS = 3
HIST = PASS_BATCHES * P
SLICE = HIST // NS
ZB = 2048


def _body(w_hbm, off_hbm, out_hbm, hist,
          dx_a, dx_b, dx_c, dy_a, dy_b, dy_c, w_a, w_b, w_c,
          w1_a, w1_b, w1_c, idx_a, idx_b, idx_c, zero_v, xf_buf,
          sin_a, sin_b, sin_c, ssc_a, ssc_b, ssc_c):
    c = lax.axis_index("c")
    s = lax.axis_index("s")
    b_in_pass = s // TILES_PER_BATCH
    part = s % TILES_PER_BATCH
    row_base = part * ROWS_PER_TILE
    myslice = s * SLICE
    lanes = lax.iota(jnp.int32, L)

    dx_r = (dx_a, dx_b, dx_c)
    dy_r = (dy_a, dy_b, dy_c)
    w_r = (w_a, w_b, w_c)
    w1_r = (w1_a, w1_b, w1_c)
    idx_r = (idx_a, idx_b, idx_c)
    sin_r = (sin_a, sin_b, sin_c)
    ssc_r = (ssc_a, ssc_b, ssc_c)

    def zinit(i, carry):
        zero_v[pl.ds(i * L, L)] = jnp.zeros((L,), jnp.float32)
        return carry

    lax.fori_loop(0, ZB // L, zinit, 0)

    def xinit(j, carry):
        xf_buf[pl.ds(j * L, L)] = (lanes + j * L).astype(jnp.float32)
        return carry

    lax.fori_loop(0, W // L, xinit, 0)

    def pass_body(pidx, pcarry):
        b_global = c * BATCHES_PER_CORE + pidx * PASS_BATCHES + b_in_pass
        for q in range(SLICE // ZB):
            pltpu.sync_copy(zero_v, hist.at[pl.ds(myslice + q * ZB, ZB)])
        plsc.subcore_barrier()

        hist_off = b_in_pass * P

        def fire_inputs(ci, sl):
            y0 = row_base + ci * RCH
            pltpu.async_copy(
                w_hbm.at[b_global, pl.ds(y0, RCH), :], w_r[sl], sin_r[sl])
            pltpu.async_copy(
                off_hbm.at[2 * b_global, pl.ds(y0, RCH), :], dx_r[sl],
                sin_r[sl])
            pltpu.async_copy(
                off_hbm.at[2 * b_global + 1, pl.ds(y0, RCH), :], dy_r[sl],
                sin_r[sl])

        def wait_inputs(sl):
            src = w_hbm.at[b_global, pl.ds(0, RCH), :]
            pltpu.make_async_copy(src, w_r[sl], sin_r[sl]).wait()
            pltpu.make_async_copy(src, dx_r[sl], sin_r[sl]).wait()
            pltpu.make_async_copy(src, dy_r[sl], sin_r[sl]).wait()

        def wait_scatter(sl):
            pltpu.make_async_copy(
                w1_r[sl], hist.at[idx_r[sl]], ssc_r[sl]).wait()

        fire_inputs(0, 0)

        def chunk_body(ci, carry):
            for k in range(SLOTS):

                @pl.when(ci % SLOTS == k)
                def _process(k=k):
                    nxt = (k + 1) % SLOTS

                    @pl.when(ci + 1 < NCHUNK)
                    def _prefetch():
                        @pl.when(ci >= 2)
                        def _drain():
                            wait_scatter(nxt)

                        fire_inputs(ci + 1, nxt)

                    wait_inputs(k)
                    y0 = row_base + ci * RCH
                    dx_cur = dx_r[k]
                    dy_cur = dy_r[k]
                    w_cur = w_r[k]
                    w1_cur = w1_r[k]
                    idx_cur = idx_r[k]
                    base = hist_off + lax.shift_left(y0, 9)
                    yfs = [(y0 + r).astype(jnp.float32) for r in range(RCH)]

                    def vec_body(j, rcarry):
                        xf = xf_buf[pl.ds(j * L, L)]
                        for r in range(RCH):
                            dxv = dx_cur[r, pl.ds(j * L, L)]
                            dyv = dy_cur[r, pl.ds(j * L, L)]
                            wv = w_cur[r, pl.ds(j * L, L)]
                            vx = (xf + R * dxv + MAGIC) - MAGIC
                            vy = (yfs[r] + R * dyv + MAGIC) - MAGIC
                            vx = jnp.minimum(jnp.maximum(vx, 0.0), W - 1.0)
                            vy = jnp.minimum(jnp.maximum(vy, 0.0), H - 1.0)
                            idx = (lax.shift_left(vy.astype(jnp.int32), 9)
                                   + vx.astype(jnp.int32) + hist_off)
                            o = r * W + j * L
                            idx_cur[pl.ds(o, L)] = idx
                            w1_cur[pl.ds(o, L)] = wv
                        return rcarry

                    lax.fori_loop(0, W // L, vec_body, 0)
                    pltpu.async_copy(
                        w1_cur, hist.at[idx_cur], ssc_r[k], add=True)

            return carry

        lax.fori_loop(0, NCHUNK, chunk_body, 0)
        for sl in range(SLOTS):
            wait_scatter(sl)
        plsc.subcore_barrier()

        pass_out = (c * BATCHES_PER_CORE + pidx * PASS_BATCHES) * P
        pltpu.sync_copy(hist.at[pl.ds(myslice, SLICE)],
                        out_hbm.at[pl.ds(pass_out + myslice, SLICE)])
        return pcarry

    lax.fori_loop(0, NPASS, pass_body, 0)


def kernel(stem_keypoint_output, stem_offset_output):
    w3 = stem_keypoint_output.reshape(B, H, W)
    off3 = stem_offset_output.reshape(2 * B, H, W)
    mesh = plsc.VectorSubcoreMesh(core_axis_name="c", subcore_axis_name="s")
    out = pl.kernel(
        _body,
        out_type=jax.ShapeDtypeStruct((B * P,), jnp.float32),
        mesh=mesh,
        compiler_params=pltpu.CompilerParams(use_tc_tiling_on_sc=True),
        scratch_types=[
            pltpu.VMEM_SHARED((HIST,), jnp.float32),
            pltpu.VMEM((RCH, W), jnp.float32),
            pltpu.VMEM((RCH, W), jnp.float32),
            pltpu.VMEM((RCH, W), jnp.float32),
            pltpu.VMEM((RCH, W), jnp.float32),
            pltpu.VMEM((RCH, W), jnp.float32),
            pltpu.VMEM((RCH, W), jnp.float32),
            pltpu.VMEM((RCH, W), jnp.float32),
            pltpu.VMEM((RCH, W), jnp.float32),
            pltpu.VMEM((RCH, W), jnp.float32),
            pltpu.VMEM((CH,), jnp.float32),
            pltpu.VMEM((CH,), jnp.float32),
            pltpu.VMEM((CH,), jnp.float32),
            pltpu.VMEM((CH,), jnp.int32),
            pltpu.VMEM((CH,), jnp.int32),
            pltpu.VMEM((CH,), jnp.int32),
            pltpu.VMEM((ZB,), jnp.float32),
            pltpu.VMEM((W,), jnp.float32),
            pltpu.SemaphoreType.DMA,
            pltpu.SemaphoreType.DMA,
            pltpu.SemaphoreType.DMA,
            pltpu.SemaphoreType.DMA,
            pltpu.SemaphoreType.DMA,
            pltpu.SemaphoreType.DMA,
        ],
    )(w3, off3)
    return out.reshape(B, H, W)
